# baseline (speedup 1.0000x reference)
import functools

import jax
import jax.numpy as jnp
from jax.experimental import pallas as pl
from jax.experimental.pallas import tpu as pltpu

_IGNORE = -100
_NEG = -1e30


def _cce_body(e_hbm, c_ref, t_ref, out_ref, ebuf, st_ref, cbf_ref, sem,
              *, bv, vb, vocab, chunk):
    p = pl.program_id(0)
    v = pl.program_id(1)
    rows = ebuf.shape[0]

    @pl.when(v == 0)
    def _():
        cp = pltpu.make_async_copy(
            e_hbm.at[pl.ds(p * rows, rows), :], ebuf, sem)
        cp.start()
        st_ref[:, 0:1] = jnp.full((rows, 1), _NEG, jnp.float32)
        st_ref[:, 1:2] = jnp.zeros((rows, 1), jnp.float32)
        st_ref[:, 2:3] = jnp.zeros((rows, 1), jnp.float32)
        cp.wait()

    cbf_ref[...] = c_ref[...].astype(jnp.bfloat16)

    lim = vocab - v * bv
    col = jax.lax.broadcasted_iota(jnp.int32, (chunk, bv), 1)

    for ch in range(rows // chunk):
        sl = pl.ds(ch * chunk, chunk)
        l = jax.lax.dot_general(
            ebuf[sl, :], cbf_ref[...],
            dimension_numbers=(((1,), (1,)), ((), ())),
            preferred_element_type=jnp.float32)

        t = t_ref[0, sl, :]
        st_ref[sl, 2:3] += jnp.sum(
            jnp.where(col == t - v * bv, l, 0.0), axis=1, keepdims=True)

        lm = jnp.where(col < lim, l, _NEG)
        bm = jnp.max(lm, axis=1, keepdims=True)
        m_old = st_ref[sl, 0:1]
        m_new = jnp.maximum(m_old, bm)
        p_sum = jnp.sum(jnp.exp(lm - m_new), axis=1, keepdims=True)
        st_ref[sl, 1:2] = st_ref[sl, 1:2] * jnp.exp(m_old - m_new) + p_sum
        st_ref[sl, 0:1] = m_new

        @pl.when(v == vb - 1)
        def _(sl=sl):
            t_fin = t_ref[0, sl, :]
            nll = (st_ref[sl, 0:1] + jnp.log(st_ref[sl, 1:2])
                   - st_ref[sl, 2:3])
            out_ref[sl, :] = jnp.where(t_fin != _IGNORE, nll, 0.0)


def _flash_cce(e_bf, c, t3, *, bv, chunk):
    ncore, rows, _ = t3.shape
    n_tok = ncore * rows
    d = e_bf.shape[1]
    vocab = c.shape[0]
    vb = pl.cdiv(vocab, bv)
    return pl.pallas_call(
        functools.partial(_cce_body, bv=bv, vb=vb, vocab=vocab, chunk=chunk),
        out_shape=jax.ShapeDtypeStruct((n_tok, 1), jnp.float32),
        grid=(ncore, vb),
        in_specs=[
            pl.BlockSpec(memory_space=pl.ANY),
            pl.BlockSpec((bv, d), lambda p, v: (v, 0)),
            pl.BlockSpec((1, rows, 1), lambda p, v: (p, 0, 0)),
        ],
        out_specs=pl.BlockSpec((rows, 1), lambda p, v: (p, 0)),
        scratch_shapes=[
            pltpu.VMEM((rows, d), jnp.bfloat16),
            pltpu.VMEM((rows, 3), jnp.float32),
            pltpu.VMEM((bv, d), jnp.bfloat16),
            pltpu.SemaphoreType.DMA,
        ],
        compiler_params=pltpu.CompilerParams(
            dimension_semantics=("parallel", "arbitrary"),
            vmem_limit_bytes=64 * 1024 * 1024,
        ),
        name="flash_cce",
    )(e_bf, c, t3)


def kernel(e, c, targets):
    n_tok, _ = e.shape
    ncore = 2
    rows = n_tok // ncore
    e_bf = e.astype(jnp.bfloat16)
    t3 = targets.astype(jnp.int32).reshape(ncore, rows, 1)
    nll = _flash_cce(e_bf, c, t3, bv=1024, chunk=512)
    valid = targets != _IGNORE
    n_valid = jnp.maximum(valid.sum(), 1).astype(jnp.float32)
    return jnp.sum(nll) / n_valid


# speedup vs baseline: 1.3027x; 1.3027x over previous
import functools

import jax
import jax.numpy as jnp
from jax.experimental import pallas as pl
from jax.experimental.pallas import tpu as pltpu

_IGNORE = -100
_NEG = -1e30


def _cce_body(e_hbm, c_ref, t_ref, out_ref, ebuf, m_ref, s_ref, g_ref,
              cbf_ref, esem, *, bv, vb, vocab, chunk):
    v = pl.program_id(0)
    rows = ebuf.shape[1]
    nch = rows // chunk

    @pl.when(v == 0)
    def _():
        ecp = pltpu.make_async_copy(e_hbm, ebuf, esem)
        ecp.start()
        m_ref[...] = jnp.full((1, rows), _NEG, jnp.float32)
        s_ref[...] = jnp.zeros((1, rows), jnp.float32)
        g_ref[...] = jnp.zeros((1, rows), jnp.float32)
        ecp.wait()

    cbf_ref[...] = c_ref[...].astype(jnp.bfloat16)

    lim = vocab - v * bv
    col = jax.lax.broadcasted_iota(jnp.int32, (bv, chunk), 0)

    def update(masked):
        totals = []
        for ch in range(nch):
            sl = pl.ds(ch * chunk, chunk)
            l = jax.lax.dot_general(
                cbf_ref[...], ebuf[:, sl],
                dimension_numbers=(((1,), (0,)), ((), ())),
                preferred_element_type=jnp.float32)
            t_sl = t_ref[0:1, sl]
            g_c = jnp.sum(jnp.where(col == t_sl - v * bv, l, 0.0),
                          axis=0, keepdims=True)
            if masked:
                l = jnp.where(col < lim, l, _NEG)
            bm = jnp.max(l, axis=0, keepdims=True)
            m_old = m_ref[0:1, sl]
            m_new = jnp.maximum(m_old, bm)
            ps = jnp.sum(jnp.exp(l - m_new), axis=0, keepdims=True)
            s_new = s_ref[0:1, sl] * jnp.exp(m_old - m_new) + ps
            g_new = g_ref[0:1, sl] + g_c
            m_ref[0:1, sl] = m_new
            s_ref[0:1, sl] = s_new
            g_ref[0:1, sl] = g_new
            if masked:
                nll = m_new + jnp.log(s_new) - g_new
                nll = jnp.where(t_sl != _IGNORE, nll, 0.0)
                totals.append(jnp.sum(nll, axis=1, keepdims=True))
        if masked:
            out_ref[...] = functools.reduce(lambda a, b: a + b, totals)

    @pl.when(v < vb - 1)
    def _():
        update(False)

    @pl.when(v == vb - 1)
    def _():
        update(True)


def _flash_cce(e_bf_t, c, t2, *, bv, chunk):
    d, rows = e_bf_t.shape
    vocab = c.shape[0]
    vb = pl.cdiv(vocab, bv)
    return pl.pallas_call(
        functools.partial(_cce_body, bv=bv, vb=vb, vocab=vocab, chunk=chunk),
        out_shape=jax.ShapeDtypeStruct((1, 1), jnp.float32),
        grid=(vb,),
        in_specs=[
            pl.BlockSpec(memory_space=pl.ANY),
            pl.BlockSpec((bv, d), lambda v: (v, 0)),
            pl.BlockSpec((1, rows), lambda v: (0, 0)),
        ],
        out_specs=pl.BlockSpec((1, 1), lambda v: (0, 0)),
        scratch_shapes=[
            pltpu.VMEM((d, rows), jnp.bfloat16),
            pltpu.VMEM((1, rows), jnp.float32),
            pltpu.VMEM((1, rows), jnp.float32),
            pltpu.VMEM((1, rows), jnp.float32),
            pltpu.VMEM((bv, d), jnp.bfloat16),
            pltpu.SemaphoreType.DMA,
        ],
        compiler_params=pltpu.CompilerParams(
            dimension_semantics=("arbitrary",),
            vmem_limit_bytes=64 * 1024 * 1024,
        ),
        name="flash_cce",
    )(e_bf_t, c, t2)


def kernel(e, c, targets):
    e_bf_t = e.astype(jnp.bfloat16).T
    t2 = targets.astype(jnp.int32).reshape(1, -1)
    loss_sum = _flash_cce(e_bf_t, c, t2, bv=512, chunk=1024)
    valid = targets != _IGNORE
    n_valid = jnp.maximum(valid.sum(), 1).astype(jnp.float32)
    return loss_sum[0, 0] / n_valid


# speedup vs baseline: 1.3152x; 1.0096x over previous
import functools

import jax
import jax.numpy as jnp
from jax.experimental import pallas as pl
from jax.experimental.pallas import tpu as pltpu

_IGNORE = -100
_NEG = -1e30


def _cce_body(e_hbm, c_ref, t_ref, out_ref, ebuf, m_ref, s_ref, g_ref,
              cbf_ref, esem, *, bv, vb, vocab, chunk):
    v = pl.program_id(0)
    rows = ebuf.shape[1]
    nch = rows // chunk

    @pl.when(v == 0)
    def _():
        ecp = pltpu.make_async_copy(e_hbm, ebuf, esem)
        ecp.start()
        m_ref[...] = jnp.full((1, rows), _NEG, jnp.float32)
        s_ref[...] = jnp.zeros((1, rows), jnp.float32)
        g_ref[...] = jnp.zeros((1, rows), jnp.float32)
        ecp.wait()

    cbf_ref[...] = c_ref[...].astype(jnp.bfloat16)

    lim = vocab - v * bv
    col = jax.lax.broadcasted_iota(jnp.int32, (bv, chunk), 0)

    def update(masked):
        totals = []
        for ch in range(nch):
            sl = pl.ds(ch * chunk, chunk)
            l = jax.lax.dot_general(
                cbf_ref[...], ebuf[:, sl],
                dimension_numbers=(((1,), (0,)), ((), ())),
                preferred_element_type=jnp.float32)
            t_sl = t_ref[0:1, sl]
            g_c = jnp.sum(jnp.where(col == t_sl - v * bv, l, 0.0),
                          axis=0, keepdims=True)
            if masked:
                l = jnp.where(col < lim, l, _NEG)
            bm = jnp.max(l, axis=0, keepdims=True)
            m_old = m_ref[0:1, sl]
            m_new = jnp.maximum(m_old, bm)
            ps = jnp.sum(jnp.exp(l - m_new), axis=0, keepdims=True)
            s_new = s_ref[0:1, sl] * jnp.exp(m_old - m_new) + ps
            g_new = g_ref[0:1, sl] + g_c
            m_ref[0:1, sl] = m_new
            s_ref[0:1, sl] = s_new
            g_ref[0:1, sl] = g_new
            if masked:
                nll = m_new + jnp.log(s_new) - g_new
                nll = jnp.where(t_sl != _IGNORE, nll, 0.0)
                totals.append(jnp.sum(nll, axis=1, keepdims=True))
        if masked:
            out_ref[...] = functools.reduce(lambda a, b: a + b, totals)

    @pl.when(v < vb - 1)
    def _():
        update(False)

    @pl.when(v == vb - 1)
    def _():
        update(True)


def _flash_cce(e_bf_t, c, t2, *, bv, chunk):
    d, rows = e_bf_t.shape
    vocab = c.shape[0]
    vb = pl.cdiv(vocab, bv)
    return pl.pallas_call(
        functools.partial(_cce_body, bv=bv, vb=vb, vocab=vocab, chunk=chunk),
        out_shape=jax.ShapeDtypeStruct((1, 1), jnp.float32),
        grid=(vb,),
        in_specs=[
            pl.BlockSpec(memory_space=pl.ANY),
            pl.BlockSpec((bv, d), lambda v: (v, 0)),
            pl.BlockSpec((1, rows), lambda v: (0, 0)),
        ],
        out_specs=pl.BlockSpec((1, 1), lambda v: (0, 0)),
        scratch_shapes=[
            pltpu.VMEM((d, rows), jnp.bfloat16),
            pltpu.VMEM((1, rows), jnp.float32),
            pltpu.VMEM((1, rows), jnp.float32),
            pltpu.VMEM((1, rows), jnp.float32),
            pltpu.VMEM((bv, d), jnp.bfloat16),
            pltpu.SemaphoreType.DMA,
        ],
        compiler_params=pltpu.CompilerParams(
            dimension_semantics=("arbitrary",),
            vmem_limit_bytes=64 * 1024 * 1024,
        ),
        name="flash_cce",
    )(e_bf_t, c, t2)


def kernel(e, c, targets):
    e_bf_t = e.astype(jnp.bfloat16).T
    t2 = targets.astype(jnp.int32).reshape(1, -1)
    loss_sum = _flash_cce(e_bf_t, c, t2, bv=512, chunk=2048)
    valid = targets != _IGNORE
    n_valid = jnp.maximum(valid.sum(), 1).astype(jnp.float32)
    return loss_sum[0, 0] / n_valid
